# baseline (speedup 1.0000x reference)
import functools

import jax
import jax.numpy as jnp
from jax import lax
from jax.experimental import pallas as pl
from jax.experimental.pallas import tpu as pltpu
from jax.experimental.pallas import tpu_sc as plsc

N = 10000
D = 128
E = 320000
NC = 2
NS = 16
K = 80
EPT = E // NS
NCHUNK = EPT // K
STRIPE = 632
LAST = N - (NS - 1) * STRIPE
ZB = 64
BN = 1000
GN = N // BN

assert NCHUNK % 2 == 0 and EPT % K == 0 and K % 8 == 0 and K <= 128
assert STRIPE % 8 == 0 and LAST % 8 == 0 and 0 < LAST <= STRIPE


def _dot(a, b):
    return lax.dot_general(a, b, (((1,), (0,)), ((), ())),
                           precision=lax.Precision.HIGHEST,
                           preferred_element_type=jnp.float32)


def _zero_stripe(s, zbuf, acc):
    def _zero_rows(r0, nrows):
        for q in range(nrows // ZB):
            pltpu.sync_copy(zbuf.at[pl.ds(0, ZB)],
                            acc.at[pl.ds(r0 + q * ZB, ZB)])
        rem = nrows % ZB
        if rem:
            pltpu.sync_copy(zbuf.at[pl.ds(0, rem)],
                            acc.at[pl.ds(r0 + (nrows // ZB) * ZB, rem)])

    r0 = s * STRIPE

    @pl.when(s < NS - 1)
    def _full():
        _zero_rows(r0, STRIPE)

    @pl.when(s == NS - 1)
    def _last():
        _zero_rows(r0, LAST)


def _copy_out(c, s, acc, out_hbm):
    r0 = s * STRIPE

    @pl.when(s < NS - 1)
    def _full():
        pltpu.sync_copy(acc.at[pl.ds(r0, STRIPE)],
                        out_hbm.at[pl.ds(c * N + r0, STRIPE)])

    @pl.when(s == NS - 1)
    def _last():
        pltpu.sync_copy(acc.at[pl.ds(r0, LAST)],
                        out_hbm.at[pl.ds(c * N + r0, LAST)])


@functools.cache
def _make_sc_segment_sum():
    mesh = plsc.VectorSubcoreMesh(core_axis_name="c", subcore_axis_name="s")

    def body(y_hbm, row_hbm, col_hbm, out_hbm,
             cidx0, cidx1, ridx0, ridx1, rows0, rows1, acc,
             gsem0, gsem1, isem0, isem1):
        c = lax.axis_index("c")
        s = lax.axis_index("s")
        base_e = c * E + s * EPT

        zvec = jnp.zeros((16,), jnp.float32)

        @pl.loop(0, ZB)
        def _zrow(r):
            for q in range(D // 16):
                rows0[r, pl.ds(q * 16, 16)] = zvec

        _zero_stripe(s, rows0, acc)
        plsc.subcore_barrier()

        cidx = (cidx0, cidx1)
        ridx = (ridx0, ridx1)
        rows = (rows0, rows1)
        gsem = (gsem0, gsem1)
        isem = (isem0, isem1)

        def load_idx(j, b):
            st = base_e + j * K
            dc = pltpu.async_copy(col_hbm.at[pl.ds(st, K)], cidx[b], isem[b])
            dr = pltpu.async_copy(row_hbm.at[pl.ds(st, K)], ridx[b], isem[b])
            return dc, dr

        def wait_idx(b):
            pltpu.make_async_copy(col_hbm.at[pl.ds(0, K)], cidx[b], isem[b]).wait()
            pltpu.make_async_copy(row_hbm.at[pl.ds(0, K)], ridx[b], isem[b]).wait()

        def start_gather(b):
            pltpu.async_copy(y_hbm.at[cidx[b]], rows[b], gsem[b])

        def wait_gather(b):
            pltpu.make_async_copy(y_hbm.at[cidx[b]], rows[b], gsem[b]).wait()

        dc, dr = load_idx(0, 0)
        dc.wait()
        dr.wait()
        start_gather(0)
        load_idx(1, 1)

        @pl.loop(0, NCHUNK, step=2)
        def _chunks(jb):
            for b in (0, 1):
                j = jb + b
                nb = 1 - b
                wait_idx(nb)
                wait_gather(b)
                start_gather(nb)
                pltpu.sync_copy(rows[b], acc.at[ridx[b]], add=True)
                jn2 = jnp.minimum(j + 2, NCHUNK - 1)
                load_idx(jn2, b)

        wait_gather(0)
        wait_idx(1)

        plsc.subcore_barrier()
        _copy_out(c, s, acc, out_hbm)

    return pl.kernel(
        body,
        out_type=jax.ShapeDtypeStruct((2 * N, D), jnp.float32),
        mesh=mesh,
        scratch_types=[
            pltpu.VMEM((K,), jnp.int32),
            pltpu.VMEM((K,), jnp.int32),
            pltpu.VMEM((K,), jnp.int32),
            pltpu.VMEM((K,), jnp.int32),
            pltpu.VMEM((K, D), jnp.float32),
            pltpu.VMEM((K, D), jnp.float32),
            pltpu.VMEM_SHARED((N, D), jnp.float32),
            pltpu.SemaphoreType.DMA,
            pltpu.SemaphoreType.DMA,
            pltpu.SemaphoreType.DMA,
            pltpu.SemaphoreType.DMA,
        ],
    )


@functools.cache
def _make_sc_degree():
    mesh = plsc.VectorSubcoreMesh(core_axis_name="c", subcore_axis_name="s")

    def body(row_hbm, out_hbm, ridx0, ridx1, ones_buf, acc, isem0, isem1):
        c = lax.axis_index("c")
        s = lax.axis_index("s")
        base_e = c * E + s * EPT

        zvec = jnp.zeros((16,), jnp.float32)

        @pl.loop(0, ZB)
        def _zrow(r):
            for q in range(D // 16):
                ones_buf[r, pl.ds(q * 16, 16)] = zvec

        _zero_stripe(s, ones_buf, acc)
        ovec = jnp.ones((16,), jnp.float32)

        @pl.loop(0, K)
        def _orow(r):
            for q in range(D // 16):
                ones_buf[r, pl.ds(q * 16, 16)] = ovec

        plsc.subcore_barrier()

        ridx = (ridx0, ridx1)
        isem = (isem0, isem1)

        def load_idx(j, b):
            pltpu.async_copy(row_hbm.at[pl.ds(base_e + j * K, K)],
                             ridx[b], isem[b])

        def wait_idx(b):
            pltpu.make_async_copy(row_hbm.at[pl.ds(0, K)], ridx[b], isem[b]).wait()

        load_idx(0, 0)
        load_idx(1, 1)

        @pl.loop(0, NCHUNK, step=2)
        def _chunks(jb):
            for b in (0, 1):
                j = jb + b
                wait_idx(b)
                pltpu.sync_copy(ones_buf, acc.at[ridx[b]], add=True)
                jn2 = jnp.minimum(j + 2, NCHUNK - 1)
                load_idx(jn2, b)

        wait_idx(0)
        wait_idx(1)

        plsc.subcore_barrier()
        _copy_out(c, s, acc, out_hbm)

    return pl.kernel(
        body,
        out_type=jax.ShapeDtypeStruct((2 * N, D), jnp.float32),
        mesh=mesh,
        scratch_types=[
            pltpu.VMEM((K,), jnp.int32),
            pltpu.VMEM((K,), jnp.int32),
            pltpu.VMEM((K, D), jnp.float32),
            pltpu.VMEM_SHARED((N, D), jnp.float32),
            pltpu.SemaphoreType.DMA,
            pltpu.SemaphoreType.DMA,
        ],
    )


def _tc1(x, wn0, ws0, b0, interpret=False):
    def body(x_ref, wn_ref, ws_ref, b_ref, y_ref, base_ref):
        xb = x_ref[...]
        y_ref[...] = _dot(xb, wn_ref[0])
        base_ref[...] = _dot(xb, ws_ref[0]) + b_ref[0]

    return pl.pallas_call(
        body,
        grid=(2, GN),
        in_specs=[
            pl.BlockSpec((BN, D), lambda g, i: (i, 0)),
            pl.BlockSpec((1, D, D), lambda g, i: (g, 0, 0)),
            pl.BlockSpec((1, D, D), lambda g, i: (g, 0, 0)),
            pl.BlockSpec((1, 1, D), lambda g, i: (g, 0, 0)),
        ],
        out_specs=[
            pl.BlockSpec((BN, D), lambda g, i: (g * GN + i, 0)),
            pl.BlockSpec((BN, D), lambda g, i: (g * GN + i, 0)),
        ],
        out_shape=[
            jax.ShapeDtypeStruct((2 * N, D), jnp.float32),
            jax.ShapeDtypeStruct((2 * N, D), jnp.float32),
        ],
        interpret=interpret,
    )(x, wn0, ws0, b0)


def _tc2(s0, cnt, base0, wn1, ws1, b1, interpret=False):
    def body(s0_ref, cnt_ref, base0_ref, wn_ref, ws_ref, b_ref,
             y_ref, base_ref):
        nei = s0_ref[...] / (cnt_ref[...] + 1e-12)
        h0 = jnp.maximum(base0_ref[...] + nei, 0.0)
        y_ref[...] = _dot(h0, wn_ref[0])
        base_ref[...] = _dot(h0, ws_ref[0]) + b_ref[0]

    return pl.pallas_call(
        body,
        grid=(2, GN),
        in_specs=[
            pl.BlockSpec((BN, D), lambda g, i: (g * GN + i, 0)),
            pl.BlockSpec((BN, D), lambda g, i: (g * GN + i, 0)),
            pl.BlockSpec((BN, D), lambda g, i: (g * GN + i, 0)),
            pl.BlockSpec((1, D, D), lambda g, i: (g, 0, 0)),
            pl.BlockSpec((1, D, D), lambda g, i: (g, 0, 0)),
            pl.BlockSpec((1, 1, D), lambda g, i: (g, 0, 0)),
        ],
        out_specs=[
            pl.BlockSpec((BN, D), lambda g, i: (g * GN + i, 0)),
            pl.BlockSpec((BN, D), lambda g, i: (g * GN + i, 0)),
        ],
        out_shape=[
            jax.ShapeDtypeStruct((2 * N, D), jnp.float32),
            jax.ShapeDtypeStruct((2 * N, D), jnp.float32),
        ],
        interpret=interpret,
    )(s0, cnt, base0, wn1, ws1, b1)


def _tc3(base1, s1, cnt, alpha, interpret=False):
    def body(b1s_ref, b1a_ref, s1s_ref, s1a_ref, cs_ref, ca_ref, a_ref,
             out_ref):
        wgt = 1.0 / (1.0 + jnp.exp(-a_ref[0, 0]))
        hs = jnp.maximum(b1s_ref[...] + s1s_ref[...] / (cs_ref[...] + 1e-12),
                         0.0)
        ha = jnp.maximum(b1a_ref[...] + s1a_ref[...] / (ca_ref[...] + 1e-12),
                         0.0)
        out_ref[...] = wgt * hs + (1.0 - wgt) * ha

    lo = lambda i: (i, 0)
    hi = lambda i: (GN + i, 0)
    return pl.pallas_call(
        body,
        grid=(GN,),
        in_specs=[
            pl.BlockSpec((BN, D), lo),
            pl.BlockSpec((BN, D), hi),
            pl.BlockSpec((BN, D), lo),
            pl.BlockSpec((BN, D), hi),
            pl.BlockSpec((BN, D), lo),
            pl.BlockSpec((BN, D), hi),
            pl.BlockSpec(memory_space=pltpu.SMEM),
        ],
        out_specs=pl.BlockSpec((BN, D), lo),
        out_shape=jax.ShapeDtypeStruct((N, D), jnp.float32),
        interpret=interpret,
    )(base1, base1, s1, s1, cnt, cnt, alpha)


def kernel(x, edge_spatial, edge_attr, alpha,
           s0_ws, s0_bs, s0_wn, s0_bn, s1_ws, s1_bs, s1_wn, s1_bn,
           a0_ws, a0_bs, a0_wn, a0_bn, a1_ws, a1_bs, a1_wn, a1_bn):
    es = edge_spatial.astype(jnp.int32)
    ea = edge_attr.astype(jnp.int32)
    row_all = jnp.concatenate([es[0], ea[0]])
    col_all = jnp.concatenate([es[1], ea[1] + N])

    wn0 = jnp.stack([s0_wn, a0_wn])
    ws0 = jnp.stack([s0_ws, a0_ws])
    b0 = jnp.stack([s0_bs + s0_bn, a0_bs + a0_bn])[:, None, :]
    wn1 = jnp.stack([s1_wn, a1_wn])
    ws1 = jnp.stack([s1_ws, a1_ws])
    b1 = jnp.stack([s1_bs + s1_bn, a1_bs + a1_bn])[:, None, :]
    alpha2 = jnp.reshape(alpha, (1, 1)).astype(jnp.float32)

    cnt = _make_sc_degree()(row_all)
    y0, base0 = _tc1(x, wn0, ws0, b0)
    s0 = _make_sc_segment_sum()(y0, row_all, col_all)
    y1, base1 = _tc2(s0, cnt, base0, wn1, ws1, b1)
    s1 = _make_sc_segment_sum()(y1, row_all, col_all)
    return _tc3(base1, s1, cnt, alpha2)


# speedup vs baseline: 1.0113x; 1.0113x over previous
import functools

import jax
import jax.numpy as jnp
from jax import lax
from jax.experimental import pallas as pl
from jax.experimental.pallas import tpu as pltpu
from jax.experimental.pallas import tpu_sc as plsc

N = 10000
D = 128
E = 320000
NC = 2
NS = 16
K = 80
EPT = E // NS
NCHUNK = EPT // K
STRIPE = 632
LAST = N - (NS - 1) * STRIPE
ZB = 64
BN = 1000
GN = N // BN

assert NCHUNK % 2 == 0 and EPT % K == 0 and K % 8 == 0 and K <= 128
assert STRIPE % 8 == 0 and LAST % 8 == 0 and 0 < LAST <= STRIPE


def _dot(a, b):
    return lax.dot_general(a, b, (((1,), (0,)), ((), ())),
                           precision=lax.Precision.HIGHEST,
                           preferred_element_type=jnp.float32)


def _zero_stripe(s, zbuf, acc):
    def _zero_rows(r0, nrows):
        for q in range(nrows // ZB):
            pltpu.sync_copy(zbuf.at[pl.ds(0, ZB)],
                            acc.at[pl.ds(r0 + q * ZB, ZB)])
        rem = nrows % ZB
        if rem:
            pltpu.sync_copy(zbuf.at[pl.ds(0, rem)],
                            acc.at[pl.ds(r0 + (nrows // ZB) * ZB, rem)])

    r0 = s * STRIPE

    @pl.when(s < NS - 1)
    def _full():
        _zero_rows(r0, STRIPE)

    @pl.when(s == NS - 1)
    def _last():
        _zero_rows(r0, LAST)


def _copy_out(c, s, acc, out_hbm):
    r0 = s * STRIPE

    @pl.when(s < NS - 1)
    def _full():
        pltpu.sync_copy(acc.at[pl.ds(r0, STRIPE)],
                        out_hbm.at[pl.ds(c * N + r0, STRIPE)])

    @pl.when(s == NS - 1)
    def _last():
        pltpu.sync_copy(acc.at[pl.ds(r0, LAST)],
                        out_hbm.at[pl.ds(c * N + r0, LAST)])


@functools.cache
def _make_sc_segment_sum():
    mesh = plsc.VectorSubcoreMesh(core_axis_name="c", subcore_axis_name="s")

    def body(y_hbm, row_hbm, col_hbm, out_hbm,
             cidx0, cidx1, ridx0, ridx1, rows0, rows1, acc,
             gsem0, gsem1, isem0, isem1):
        c = lax.axis_index("c")
        s = lax.axis_index("s")
        base_e = c * E + s * EPT

        zvec = jnp.zeros((16,), jnp.float32)

        @pl.loop(0, ZB)
        def _zrow(r):
            for q in range(D // 16):
                rows0[r, pl.ds(q * 16, 16)] = zvec

        _zero_stripe(s, rows0, acc)
        plsc.subcore_barrier()

        cidx = (cidx0, cidx1)
        ridx = (ridx0, ridx1)
        rows = (rows0, rows1)
        gsem = (gsem0, gsem1)
        isem = (isem0, isem1)

        def load_idx(j, b):
            st = base_e + j * K
            dc = pltpu.async_copy(col_hbm.at[pl.ds(st, K)], cidx[b], isem[b])
            dr = pltpu.async_copy(row_hbm.at[pl.ds(st, K)], ridx[b], isem[b])
            return dc, dr

        def wait_idx(b):
            pltpu.make_async_copy(col_hbm.at[pl.ds(0, K)], cidx[b], isem[b]).wait()
            pltpu.make_async_copy(row_hbm.at[pl.ds(0, K)], ridx[b], isem[b]).wait()

        def start_gather(b):
            pltpu.async_copy(y_hbm.at[cidx[b]], rows[b], gsem[b])

        def wait_gather(b):
            pltpu.make_async_copy(y_hbm.at[cidx[b]], rows[b], gsem[b]).wait()

        dc, dr = load_idx(0, 0)
        dc.wait()
        dr.wait()
        start_gather(0)
        load_idx(1, 1)

        @pl.loop(0, NCHUNK, step=2)
        def _chunks(jb):
            for b in (0, 1):
                j = jb + b
                nb = 1 - b
                wait_idx(nb)
                wait_gather(b)
                start_gather(nb)
                pltpu.sync_copy(rows[b], acc.at[ridx[b]], add=True)
                jn2 = jnp.minimum(j + 2, NCHUNK - 1)
                load_idx(jn2, b)

        wait_gather(0)
        wait_idx(1)

        plsc.subcore_barrier()
        _copy_out(c, s, acc, out_hbm)

    return pl.kernel(
        body,
        out_type=jax.ShapeDtypeStruct((2 * N, D), jnp.float32),
        mesh=mesh,
        scratch_types=[
            pltpu.VMEM((K,), jnp.int32),
            pltpu.VMEM((K,), jnp.int32),
            pltpu.VMEM((K,), jnp.int32),
            pltpu.VMEM((K,), jnp.int32),
            pltpu.VMEM((K, D), jnp.float32),
            pltpu.VMEM((K, D), jnp.float32),
            pltpu.VMEM_SHARED((N, D), jnp.float32),
            pltpu.SemaphoreType.DMA,
            pltpu.SemaphoreType.DMA,
            pltpu.SemaphoreType.DMA,
            pltpu.SemaphoreType.DMA,
        ],
    )


@functools.cache
def _make_sc_degree():
    mesh = plsc.VectorSubcoreMesh(core_axis_name="c", subcore_axis_name="s")

    def body(row_hbm, out_hbm, ridx0, ridx1, ones_buf, acc, isem0, isem1):
        c = lax.axis_index("c")
        s = lax.axis_index("s")
        base_e = c * E + s * EPT

        zvec = jnp.zeros((16,), jnp.float32)

        @pl.loop(0, ZB)
        def _zrow(r):
            for q in range(D // 16):
                ones_buf[r, pl.ds(q * 16, 16)] = zvec

        _zero_stripe(s, ones_buf, acc)
        ovec = jnp.ones((16,), jnp.float32)

        @pl.loop(0, K)
        def _orow(r):
            for q in range(D // 16):
                ones_buf[r, pl.ds(q * 16, 16)] = ovec

        plsc.subcore_barrier()

        ridx = (ridx0, ridx1)
        isem = (isem0, isem1)

        def load_idx(j, b):
            pltpu.async_copy(row_hbm.at[pl.ds(base_e + j * K, K)],
                             ridx[b], isem[b])

        def wait_idx(b):
            pltpu.make_async_copy(row_hbm.at[pl.ds(0, K)], ridx[b], isem[b]).wait()

        load_idx(0, 0)
        load_idx(1, 1)

        @pl.loop(0, NCHUNK, step=2)
        def _chunks(jb):
            for b in (0, 1):
                j = jb + b
                wait_idx(b)
                pltpu.sync_copy(ones_buf, acc.at[ridx[b]], add=True)
                jn2 = jnp.minimum(j + 2, NCHUNK - 1)
                load_idx(jn2, b)

        wait_idx(0)
        wait_idx(1)

        plsc.subcore_barrier()
        _copy_out(c, s, acc, out_hbm)

    return pl.kernel(
        body,
        out_type=jax.ShapeDtypeStruct((2 * N, D), jnp.float32),
        mesh=mesh,
        scratch_types=[
            pltpu.VMEM((K,), jnp.int32),
            pltpu.VMEM((K,), jnp.int32),
            pltpu.VMEM((K, D), jnp.float32),
            pltpu.VMEM_SHARED((N, D), jnp.float32),
            pltpu.SemaphoreType.DMA,
            pltpu.SemaphoreType.DMA,
        ],
    )


def _tc1(x, wn0, ws0, b0, interpret=False):
    def body(x_ref, wn_ref, ws_ref, b_ref, y_ref, base_ref):
        xb = x_ref[...]
        y_ref[...] = _dot(xb, wn_ref[0])
        base_ref[...] = _dot(xb, ws_ref[0]) + b_ref[0]

    return pl.pallas_call(
        body,
        grid=(2, GN),
        in_specs=[
            pl.BlockSpec((BN, D), lambda g, i: (i, 0)),
            pl.BlockSpec((1, D, D), lambda g, i: (g, 0, 0)),
            pl.BlockSpec((1, D, D), lambda g, i: (g, 0, 0)),
            pl.BlockSpec((1, 1, D), lambda g, i: (g, 0, 0)),
        ],
        out_specs=[
            pl.BlockSpec((BN, D), lambda g, i: (g * GN + i, 0)),
            pl.BlockSpec((BN, D), lambda g, i: (g * GN + i, 0)),
        ],
        out_shape=[
            jax.ShapeDtypeStruct((2 * N, D), jnp.float32),
            jax.ShapeDtypeStruct((2 * N, D), jnp.float32),
        ],
        interpret=interpret,
    )(x, wn0, ws0, b0)


def _tc2(s0, cnt, base0, interpret=False):
    def body(s0_ref, cnt_ref, base0_ref, h_ref):
        nei = s0_ref[...] / (cnt_ref[...] + 1e-12)
        h_ref[...] = jnp.maximum(base0_ref[...] + nei, 0.0)

    spec = pl.BlockSpec((BN, D), lambda i: (i, 0))
    return pl.pallas_call(
        body,
        grid=(2 * GN,),
        in_specs=[spec, spec, spec],
        out_specs=spec,
        out_shape=jax.ShapeDtypeStruct((2 * N, D), jnp.float32),
        interpret=interpret,
    )(s0, cnt, base0)


def _tcb1(h0, ws1, b1, interpret=False):
    def body(h_ref, ws_ref, b_ref, base_ref):
        base_ref[...] = _dot(h_ref[...], ws_ref[0]) + b_ref[0]

    return pl.pallas_call(
        body,
        grid=(2, GN),
        in_specs=[
            pl.BlockSpec((BN, D), lambda g, i: (g * GN + i, 0)),
            pl.BlockSpec((1, D, D), lambda g, i: (g, 0, 0)),
            pl.BlockSpec((1, 1, D), lambda g, i: (g, 0, 0)),
        ],
        out_specs=pl.BlockSpec((BN, D), lambda g, i: (g * GN + i, 0)),
        out_shape=jax.ShapeDtypeStruct((2 * N, D), jnp.float32),
        interpret=interpret,
    )(h0, ws1, b1)


def _tc3(base1, s1h, cnt, wn1, alpha, interpret=False):
    def body(b1s_ref, b1a_ref, s1s_ref, s1a_ref, cs_ref, ca_ref, wn_ref,
             a_ref, out_ref):
        wgt = 1.0 / (1.0 + jnp.exp(-a_ref[0, 0]))
        ns = _dot(s1s_ref[...], wn_ref[0]) / (cs_ref[...] + 1e-12)
        na = _dot(s1a_ref[...], wn_ref[1]) / (ca_ref[...] + 1e-12)
        hs = jnp.maximum(b1s_ref[...] + ns, 0.0)
        ha = jnp.maximum(b1a_ref[...] + na, 0.0)
        out_ref[...] = wgt * hs + (1.0 - wgt) * ha

    lo = lambda i: (i, 0)
    hi = lambda i: (GN + i, 0)
    return pl.pallas_call(
        body,
        grid=(GN,),
        in_specs=[
            pl.BlockSpec((BN, D), lo),
            pl.BlockSpec((BN, D), hi),
            pl.BlockSpec((BN, D), lo),
            pl.BlockSpec((BN, D), hi),
            pl.BlockSpec((BN, D), lo),
            pl.BlockSpec((BN, D), hi),
            pl.BlockSpec((2, D, D), lambda i: (0, 0, 0)),
            pl.BlockSpec(memory_space=pltpu.SMEM),
        ],
        out_specs=pl.BlockSpec((BN, D), lo),
        out_shape=jax.ShapeDtypeStruct((N, D), jnp.float32),
        interpret=interpret,
    )(base1, base1, s1h, s1h, cnt, cnt, wn1, alpha)


def kernel(x, edge_spatial, edge_attr, alpha,
           s0_ws, s0_bs, s0_wn, s0_bn, s1_ws, s1_bs, s1_wn, s1_bn,
           a0_ws, a0_bs, a0_wn, a0_bn, a1_ws, a1_bs, a1_wn, a1_bn):
    es = edge_spatial.astype(jnp.int32)
    ea = edge_attr.astype(jnp.int32)
    row_all = jnp.concatenate([es[0], ea[0]])
    col_all = jnp.concatenate([es[1], ea[1] + N])

    wn0 = jnp.stack([s0_wn, a0_wn])
    ws0 = jnp.stack([s0_ws, a0_ws])
    b0 = jnp.stack([s0_bs + s0_bn, a0_bs + a0_bn])[:, None, :]
    wn1 = jnp.stack([s1_wn, a1_wn])
    ws1 = jnp.stack([s1_ws, a1_ws])
    b1 = jnp.stack([s1_bs + s1_bn, a1_bs + a1_bn])[:, None, :]
    alpha2 = jnp.reshape(alpha, (1, 1)).astype(jnp.float32)

    cnt = _make_sc_degree()(row_all)
    y0, base0 = _tc1(x, wn0, ws0, b0)
    s0 = _make_sc_segment_sum()(y0, row_all, col_all)
    h0 = _tc2(s0, cnt, base0)
    base1 = _tcb1(h0, ws1, b1)
    s1h = _make_sc_segment_sum()(h0, row_all, col_all)
    return _tc3(base1, s1h, cnt, wn1, alpha2)


# speedup vs baseline: 1.0458x; 1.0341x over previous
import functools

import jax
import jax.numpy as jnp
from jax import lax
from jax.experimental import pallas as pl
from jax.experimental.pallas import tpu as pltpu
from jax.experimental.pallas import tpu_sc as plsc

N = 10000
D = 128
E = 320000
NC = 2
NS = 16
K = 80
EPT = E // NS
NCHUNK = EPT // K
STRIPE = 632
LAST = N - (NS - 1) * STRIPE
ZB = 64
BN = 1000
GN = N // BN

assert NCHUNK % 2 == 0 and EPT % K == 0 and K % 8 == 0 and K <= 128
assert STRIPE % 8 == 0 and LAST % 8 == 0 and 0 < LAST <= STRIPE


def _dot(a, b):
    return lax.dot_general(a, b, (((1,), (0,)), ((), ())),
                           precision=lax.Precision.HIGHEST,
                           preferred_element_type=jnp.float32)


def _zero_stripe(s, zbuf, acc):
    def _zero_rows(r0, nrows):
        for q in range(nrows // ZB):
            pltpu.sync_copy(zbuf.at[pl.ds(0, ZB)],
                            acc.at[pl.ds(r0 + q * ZB, ZB)])
        rem = nrows % ZB
        if rem:
            pltpu.sync_copy(zbuf.at[pl.ds(0, rem)],
                            acc.at[pl.ds(r0 + (nrows // ZB) * ZB, rem)])

    r0 = s * STRIPE

    @pl.when(s < NS - 1)
    def _full():
        _zero_rows(r0, STRIPE)

    @pl.when(s == NS - 1)
    def _last():
        _zero_rows(r0, LAST)


def _copy_out(c, s, acc, out_hbm):
    r0 = s * STRIPE

    @pl.when(s < NS - 1)
    def _full():
        pltpu.sync_copy(acc.at[pl.ds(r0, STRIPE)],
                        out_hbm.at[pl.ds(c * N + r0, STRIPE)])

    @pl.when(s == NS - 1)
    def _last():
        pltpu.sync_copy(acc.at[pl.ds(r0, LAST)],
                        out_hbm.at[pl.ds(c * N + r0, LAST)])


@functools.cache
def _make_sc_segment_sum():
    mesh = plsc.VectorSubcoreMesh(core_axis_name="c", subcore_axis_name="s")

    def body(y_hbm, row_hbm, col_hbm, out_hbm,
             cidx0, cidx1, ridx0, ridx1, rows0, rows1, acc,
             gsem0, gsem1, isem0, isem1):
        c = lax.axis_index("c")
        s = lax.axis_index("s")
        base_e = c * E + s * EPT

        zvec = jnp.zeros((16,), jnp.float32)

        @pl.loop(0, ZB)
        def _zrow(r):
            for q in range(D // 16):
                rows0[r, pl.ds(q * 16, 16)] = zvec

        _zero_stripe(s, rows0, acc)
        plsc.subcore_barrier()

        cidx = (cidx0, cidx1)
        ridx = (ridx0, ridx1)
        rows = (rows0, rows1)
        gsem = (gsem0, gsem1)
        isem = (isem0, isem1)

        def load_idx(j, b):
            st = base_e + j * K
            dc = pltpu.async_copy(col_hbm.at[pl.ds(st, K)], cidx[b], isem[b])
            dr = pltpu.async_copy(row_hbm.at[pl.ds(st, K)], ridx[b], isem[b])
            return dc, dr

        def wait_idx(b):
            pltpu.make_async_copy(col_hbm.at[pl.ds(0, K)], cidx[b], isem[b]).wait()
            pltpu.make_async_copy(row_hbm.at[pl.ds(0, K)], ridx[b], isem[b]).wait()

        def start_gather(b):
            pltpu.async_copy(y_hbm.at[cidx[b]], rows[b], gsem[b])

        def wait_gather(b):
            pltpu.make_async_copy(y_hbm.at[cidx[b]], rows[b], gsem[b]).wait()

        dc, dr = load_idx(0, 0)
        dc.wait()
        dr.wait()
        start_gather(0)
        load_idx(1, 1)

        @pl.loop(0, NCHUNK, step=2)
        def _chunks(jb):
            for b in (0, 1):
                j = jb + b
                nb = 1 - b
                wait_idx(nb)
                wait_gather(b)
                start_gather(nb)
                pltpu.sync_copy(rows[b], acc.at[ridx[b]], add=True)
                jn2 = jnp.minimum(j + 2, NCHUNK - 1)
                load_idx(jn2, b)

        wait_gather(0)
        wait_idx(1)

        plsc.subcore_barrier()
        _copy_out(c, s, acc, out_hbm)

    return pl.kernel(
        body,
        out_type=jax.ShapeDtypeStruct((2 * N, D), jnp.float32),
        mesh=mesh,
        scratch_types=[
            pltpu.VMEM((K,), jnp.int32),
            pltpu.VMEM((K,), jnp.int32),
            pltpu.VMEM((K,), jnp.int32),
            pltpu.VMEM((K,), jnp.int32),
            pltpu.VMEM((K, D), jnp.float32),
            pltpu.VMEM((K, D), jnp.float32),
            pltpu.VMEM_SHARED((N, D), jnp.float32),
            pltpu.SemaphoreType.DMA,
            pltpu.SemaphoreType.DMA,
            pltpu.SemaphoreType.DMA,
            pltpu.SemaphoreType.DMA,
        ],
    )


@functools.cache
def _make_sc_degree():
    mesh = plsc.VectorSubcoreMesh(core_axis_name="c", subcore_axis_name="s")

    def body(row_hbm, out_hbm, ridx0, ridx1, ones_buf, acc, isem0, isem1):
        c = lax.axis_index("c")
        s = lax.axis_index("s")
        base_e = c * E + s * EPT

        zvec = jnp.zeros((16,), jnp.float32)

        @pl.loop(0, ZB)
        def _zrow(r):
            for q in range(D // 16):
                ones_buf[r, pl.ds(q * 16, 16)] = zvec

        _zero_stripe(s, ones_buf, acc)
        ovec = jnp.ones((16,), jnp.float32)

        @pl.loop(0, K)
        def _orow(r):
            for q in range(D // 16):
                ones_buf[r, pl.ds(q * 16, 16)] = ovec

        plsc.subcore_barrier()

        ridx = (ridx0, ridx1)
        isem = (isem0, isem1)

        def load_idx(j, b):
            pltpu.async_copy(row_hbm.at[pl.ds(base_e + j * K, K)],
                             ridx[b], isem[b])

        def wait_idx(b):
            pltpu.make_async_copy(row_hbm.at[pl.ds(0, K)], ridx[b], isem[b]).wait()

        load_idx(0, 0)
        load_idx(1, 1)

        @pl.loop(0, NCHUNK, step=2)
        def _chunks(jb):
            for b in (0, 1):
                j = jb + b
                wait_idx(b)
                pltpu.sync_copy(ones_buf, acc.at[ridx[b]], add=True)
                jn2 = jnp.minimum(j + 2, NCHUNK - 1)
                load_idx(jn2, b)

        wait_idx(0)
        wait_idx(1)

        plsc.subcore_barrier()
        _copy_out(c, s, acc, out_hbm)

    return pl.kernel(
        body,
        out_type=jax.ShapeDtypeStruct((2 * N, D), jnp.float32),
        mesh=mesh,
        scratch_types=[
            pltpu.VMEM((K,), jnp.int32),
            pltpu.VMEM((K,), jnp.int32),
            pltpu.VMEM((K, D), jnp.float32),
            pltpu.VMEM_SHARED((N, D), jnp.float32),
            pltpu.SemaphoreType.DMA,
            pltpu.SemaphoreType.DMA,
        ],
    )


def _tc_base(inp, ws, b, shared_input, interpret=False):
    def body(x_ref, ws_ref, b_ref, base_ref):
        base_ref[...] = _dot(x_ref[...], ws_ref[0]) + b_ref[0]

    in_map = ((lambda g, i: (i, 0)) if shared_input
              else (lambda g, i: (g * GN + i, 0)))
    return pl.pallas_call(
        body,
        grid=(2, GN),
        in_specs=[
            pl.BlockSpec((BN, D), in_map),
            pl.BlockSpec((1, D, D), lambda g, i: (g, 0, 0)),
            pl.BlockSpec((1, 1, D), lambda g, i: (g, 0, 0)),
        ],
        out_specs=pl.BlockSpec((BN, D), lambda g, i: (g * GN + i, 0)),
        out_shape=jax.ShapeDtypeStruct((2 * N, D), jnp.float32),
        interpret=interpret,
    )(inp, ws, b)


def _tc2(s0x, cnt, base0, wn0, interpret=False):
    def body(s0_ref, cnt_ref, base0_ref, wn_ref, h_ref):
        nei = _dot(s0_ref[...], wn_ref[0]) / (cnt_ref[...] + 1e-12)
        h_ref[...] = jnp.maximum(base0_ref[...] + nei, 0.0)

    blk = pl.BlockSpec((BN, D), lambda g, i: (g * GN + i, 0))
    return pl.pallas_call(
        body,
        grid=(2, GN),
        in_specs=[blk, blk, blk,
                  pl.BlockSpec((1, D, D), lambda g, i: (g, 0, 0))],
        out_specs=blk,
        out_shape=jax.ShapeDtypeStruct((2 * N, D), jnp.float32),
        interpret=interpret,
    )(s0x, cnt, base0, wn0)


def _tc3(base1, s1h, cnt, wn1, alpha, interpret=False):
    def body(b1s_ref, b1a_ref, s1s_ref, s1a_ref, cs_ref, ca_ref, wn_ref,
             a_ref, out_ref):
        wgt = 1.0 / (1.0 + jnp.exp(-a_ref[0, 0]))
        ns = _dot(s1s_ref[...], wn_ref[0]) / (cs_ref[...] + 1e-12)
        na = _dot(s1a_ref[...], wn_ref[1]) / (ca_ref[...] + 1e-12)
        hs = jnp.maximum(b1s_ref[...] + ns, 0.0)
        ha = jnp.maximum(b1a_ref[...] + na, 0.0)
        out_ref[...] = wgt * hs + (1.0 - wgt) * ha

    lo = lambda i: (i, 0)
    hi = lambda i: (GN + i, 0)
    return pl.pallas_call(
        body,
        grid=(GN,),
        in_specs=[
            pl.BlockSpec((BN, D), lo),
            pl.BlockSpec((BN, D), hi),
            pl.BlockSpec((BN, D), lo),
            pl.BlockSpec((BN, D), hi),
            pl.BlockSpec((BN, D), lo),
            pl.BlockSpec((BN, D), hi),
            pl.BlockSpec((2, D, D), lambda i: (0, 0, 0)),
            pl.BlockSpec(memory_space=pltpu.SMEM),
        ],
        out_specs=pl.BlockSpec((BN, D), lo),
        out_shape=jax.ShapeDtypeStruct((N, D), jnp.float32),
        interpret=interpret,
    )(base1, base1, s1h, s1h, cnt, cnt, wn1, alpha)


def kernel(x, edge_spatial, edge_attr, alpha,
           s0_ws, s0_bs, s0_wn, s0_bn, s1_ws, s1_bs, s1_wn, s1_bn,
           a0_ws, a0_bs, a0_wn, a0_bn, a1_ws, a1_bs, a1_wn, a1_bn):
    es = edge_spatial.astype(jnp.int32)
    ea = edge_attr.astype(jnp.int32)
    row_all = jnp.concatenate([es[0], ea[0]])
    col_loc = jnp.concatenate([es[1], ea[1]])
    col_glob = jnp.concatenate([es[1], ea[1] + N])

    wn0 = jnp.stack([s0_wn, a0_wn])
    ws0 = jnp.stack([s0_ws, a0_ws])
    b0 = jnp.stack([s0_bs + s0_bn, a0_bs + a0_bn])[:, None, :]
    wn1 = jnp.stack([s1_wn, a1_wn])
    ws1 = jnp.stack([s1_ws, a1_ws])
    b1 = jnp.stack([s1_bs + s1_bn, a1_bs + a1_bn])[:, None, :]
    alpha2 = jnp.reshape(alpha, (1, 1)).astype(jnp.float32)

    cnt = _make_sc_degree()(row_all)
    s0x = _make_sc_segment_sum()(x, row_all, col_loc)
    base0 = _tc_base(x, ws0, b0, True)
    h0 = _tc2(s0x, cnt, base0, wn0)
    base1 = _tc_base(h0, ws1, b1, False)
    s1h = _make_sc_segment_sum()(h0, row_all, col_glob)
    return _tc3(base1, s1h, cnt, wn1, alpha2)
